# Initial kernel scaffold; baseline (speedup 1.0000x reference)
import jax
import jax.numpy as jnp
from jax.experimental import pallas as pl


def kernel(param, centroids):
    raise NotImplementedError("write your pallas kernel here")


# speedup vs baseline: 12.6219x; 12.6219x over previous
import jax
import jax.numpy as jnp
from jax.experimental import pallas as pl
from jax.experimental.pallas import tpu as pltpu

_THRESH = 0.02
_K = 16
_BLOCK_ROWS = 256


def _quant_kernel(cs_ref, b_ref, p_ref, o_ref):
    v = p_ref[...]
    keep = jnp.abs(v) > _THRESH
    res = jnp.full(v.shape, cs_ref[0], v.dtype)
    for i in range(_K - 1):
        res = jnp.where(v > b_ref[i], cs_ref[i + 1], res)
    o_ref[...] = jnp.where(keep, res, cs_ref[_K])


def kernel(param, centroids):
    rows, cols = param.shape
    cs = jnp.sort(centroids)
    bounds = 0.5 * (cs[:-1] + cs[1:])
    zidx = jnp.sum((bounds < 0.0).astype(jnp.int32))
    cs_ext = jnp.concatenate([cs, cs[zidx][None]])
    grid = (rows // _BLOCK_ROWS,)
    return pl.pallas_call(
        _quant_kernel,
        grid=grid,
        in_specs=[
            pl.BlockSpec(memory_space=pltpu.SMEM),
            pl.BlockSpec(memory_space=pltpu.SMEM),
            pl.BlockSpec((_BLOCK_ROWS, cols), lambda i: (i, 0)),
        ],
        out_specs=pl.BlockSpec((_BLOCK_ROWS, cols), lambda i: (i, 0)),
        out_shape=jax.ShapeDtypeStruct(param.shape, param.dtype),
    )(cs_ext, bounds, param)
